# Initial kernel scaffold; baseline (speedup 1.0000x reference)
import jax
import jax.numpy as jnp
from jax.experimental import pallas as pl


def kernel(x, edge_index, batch, W1l, b1l, W1r, W2l, b2l, W2r, Wc, bc):
    raise NotImplementedError("write your pallas kernel here")


# speedup vs baseline: 5.8552x; 5.8552x over previous
import jax
import jax.numpy as jnp
from jax import lax
from jax.experimental import pallas as pl
from jax.experimental.pallas import tpu as pltpu
from jax.experimental.pallas import tpu_sc as plsc

_NC = 2
_NS = 16
_NW = _NC * _NS
_CH = 128


def _largest_div_le(n, cap):
    for cand in range(min(n, cap), 0, -1):
        if n % cand == 0:
            return cand
    return 1


def _sc_edge_aggregate(table, src, dst):
    n, d = table.shape
    e = src.shape[0]
    assert e % _CH == 0 and d % 16 == 0
    nchunk = e // _CH
    iters = -(-nchunk // _NW)
    nps = -(-n // (_NS * 8)) * 8
    npad = nps * _NS
    zr = _largest_div_le(nps, 128)
    nz = nps // zr

    scratch = [
        pltpu.VMEM((_CH,), jnp.int32),
        pltpu.VMEM((_CH,), jnp.int32),
        pltpu.VMEM((_CH, d), jnp.float32),
        pltpu.VMEM((zr, d), jnp.float32),
        pltpu.VMEM_SHARED((npad, d), jnp.float32),
        pltpu.SemaphoreType.DMA,
    ]

    def body(x_hbm, src_hbm, dst_hbm, out_hbm,
             src_v, dst_v, rows_v, zrow_v, acc_sh, sem):
        cid = lax.axis_index("c")
        sid = lax.axis_index("s")
        w = sid * _NC + cid

        zero16 = jnp.zeros((16,), jnp.float32)

        @pl.loop(0, zr)
        def _(r):
            for j in range(d // 16):
                zrow_v[r, pl.ds(16 * j, 16)] = zero16

        base = sid * nps

        @pl.loop(0, nz)
        def _(z):
            pltpu.sync_copy(zrow_v, acc_sh.at[pl.ds(base + z * zr, zr)])

        plsc.subcore_barrier()

        @pl.loop(0, iters)
        def _(i):
            c = w + i * _NW

            @pl.when(c < nchunk)
            def _():
                off = c * _CH
                pltpu.sync_copy(src_hbm.at[pl.ds(off, _CH)], src_v)
                pltpu.sync_copy(dst_hbm.at[pl.ds(off, _CH)], dst_v)
                pltpu.async_copy(x_hbm.at[src_v], rows_v, sem).wait()
                pltpu.sync_copy(rows_v, acc_sh.at[dst_v], add=True)

        plsc.subcore_barrier()
        obase = cid * npad + base
        pltpu.sync_copy(acc_sh.at[pl.ds(base, nps)], out_hbm.at[pl.ds(obase, nps)])

    mesh = plsc.VectorSubcoreMesh(core_axis_name="c", subcore_axis_name="s")
    f = pl.kernel(body,
                  out_type=jax.ShapeDtypeStruct((_NC * npad, d), jnp.float32),
                  mesh=mesh, scratch_types=scratch)
    return f(table, src, dst), npad


def _sc_degree_count(dst, n):
    e = dst.shape[0]
    assert e % _CH == 0
    nchunk = e // _CH
    iters = -(-nchunk // _NW)
    nps = -(-n // (_NS * 8)) * 8
    npad = nps * _NS
    zr = _largest_div_le(nps, 128)
    nz = nps // zr

    scratch = [
        pltpu.VMEM((_CH,), jnp.int32),
        pltpu.VMEM((zr, 128), jnp.float32),
        pltpu.VMEM((_CH, 128), jnp.float32),
        pltpu.VMEM_SHARED((npad, 128), jnp.float32),
    ]

    def body(dst_hbm, cnt_hbm, dst_v, zc_v, ones_v, cacc_sh):
        cid = lax.axis_index("c")
        sid = lax.axis_index("s")
        w = sid * _NC + cid

        zero16 = jnp.zeros((16,), jnp.float32)
        one16 = jnp.ones((16,), jnp.float32)

        @pl.loop(0, zr)
        def _(r):
            for j in range(8):
                zc_v[r, pl.ds(16 * j, 16)] = zero16

        @pl.loop(0, _CH)
        def _(r):
            for j in range(8):
                ones_v[r, pl.ds(16 * j, 16)] = one16

        base = sid * nps

        @pl.loop(0, nz)
        def _(z):
            pltpu.sync_copy(zc_v, cacc_sh.at[pl.ds(base + z * zr, zr)])

        plsc.subcore_barrier()

        @pl.loop(0, iters)
        def _(i):
            c = w + i * _NW

            @pl.when(c < nchunk)
            def _():
                pltpu.sync_copy(dst_hbm.at[pl.ds(c * _CH, _CH)], dst_v)
                pltpu.sync_copy(ones_v, cacc_sh.at[dst_v], add=True)

        plsc.subcore_barrier()
        obase = cid * npad + base
        pltpu.sync_copy(cacc_sh.at[pl.ds(base, nps)],
                        cnt_hbm.at[pl.ds(obase, nps)])

    mesh = plsc.VectorSubcoreMesh(core_axis_name="c", subcore_axis_name="s")
    f = pl.kernel(body,
                  out_type=jax.ShapeDtypeStruct((_NC * npad, 128), jnp.float32),
                  mesh=mesh, scratch_types=scratch)
    return f(dst)


def _sc_pool(h, batch, g):
    n, d = h.shape
    full = n // _CH
    tail = n - full * _CH
    iters = -(-full // _NW)
    assert g % 8 == 0 and g // 8 <= _NS and (tail == 0 or tail % 8 == 0)
    gps = 8
    nown = g // 8

    scratch = [
        pltpu.VMEM((_CH,), jnp.int32),
        pltpu.VMEM((_CH, d), jnp.float32),
        pltpu.VMEM((_CH, 128), jnp.float32),
        pltpu.VMEM((gps, d), jnp.float32),
        pltpu.VMEM((gps, 128), jnp.float32),
    ]
    if tail:
        scratch += [
            pltpu.VMEM((tail,), jnp.int32),
            pltpu.VMEM((tail, d), jnp.float32),
        ]
    scratch += [
        pltpu.VMEM_SHARED((g, d), jnp.float32),
        pltpu.VMEM_SHARED((g, 128), jnp.float32),
        pltpu.SemaphoreType.DMA,
    ]

    def body(h_hbm, b_hbm, out_hbm, cnt_hbm, *refs):
        bidx_v, rows_v, ones_v, zrow_v, zc_v = refs[:5]
        rest = refs[5:]
        if tail:
            bt_v, rowst_v = rest[0], rest[1]
            rest = rest[2:]
        acc_sh, cacc_sh, sem = rest
        cid = lax.axis_index("c")
        sid = lax.axis_index("s")
        w = sid * _NC + cid

        zero16 = jnp.zeros((16,), jnp.float32)
        one16 = jnp.ones((16,), jnp.float32)

        @pl.loop(0, gps)
        def _(r):
            for j in range(d // 16):
                zrow_v[r, pl.ds(16 * j, 16)] = zero16
            for j in range(8):
                zc_v[r, pl.ds(16 * j, 16)] = zero16

        @pl.loop(0, _CH)
        def _(r):
            for j in range(8):
                ones_v[r, pl.ds(16 * j, 16)] = one16

        base = sid * gps

        @pl.when(sid < nown)
        def _():
            pltpu.sync_copy(zrow_v, acc_sh.at[pl.ds(base, gps)])
            pltpu.sync_copy(zc_v, cacc_sh.at[pl.ds(base, gps)])

        plsc.subcore_barrier()

        @pl.loop(0, iters)
        def _(i):
            c = w + i * _NW

            @pl.when(c < full)
            def _():
                off = c * _CH
                pltpu.sync_copy(b_hbm.at[pl.ds(off, _CH)], bidx_v)
                pltpu.sync_copy(h_hbm.at[pl.ds(off, _CH)], rows_v)
                pltpu.sync_copy(rows_v, acc_sh.at[bidx_v], add=True)
                pltpu.sync_copy(ones_v, cacc_sh.at[bidx_v], add=True)

        if tail:
            @pl.when(w == _NW - 1)
            def _():
                off = full * _CH
                pltpu.sync_copy(b_hbm.at[pl.ds(off, tail)], bt_v)
                pltpu.sync_copy(h_hbm.at[pl.ds(off, tail)], rowst_v)
                pltpu.sync_copy(rowst_v, acc_sh.at[bt_v], add=True)
                pltpu.sync_copy(ones_v.at[pl.ds(0, tail)], cacc_sh.at[bt_v],
                                add=True)

        plsc.subcore_barrier()
        obase = cid * g + base

        @pl.when(sid < nown)
        def _():
            pltpu.sync_copy(acc_sh.at[pl.ds(base, gps)],
                            out_hbm.at[pl.ds(obase, gps)])
            pltpu.sync_copy(cacc_sh.at[pl.ds(base, gps)],
                            cnt_hbm.at[pl.ds(obase, gps)])

    mesh = plsc.VectorSubcoreMesh(core_axis_name="c", subcore_axis_name="s")
    f = pl.kernel(body,
                  out_type=(jax.ShapeDtypeStruct((_NC * g, d), jnp.float32),
                            jax.ShapeDtypeStruct((_NC * g, 128), jnp.float32)),
                  mesh=mesh, scratch_types=scratch)
    return f(h, batch)


def _tc_sage_linear(parts, cnts, x, wl, bias, wr, npad):
    n, d = x.shape
    br = _largest_div_le(npad, 1024)
    while br % 8 != 0:
        br //= 2
    nb = -(-n // br)
    off = npad // br

    def body(p0, p1, c0, c1, x_ref, wl_ref, wr_ref, b_ref, o_ref):
        s = p0[...] + p1[...]
        c = c0[...][:, 0:1] + c1[...][:, 0:1]
        agg = s / jnp.maximum(c, 1.0)
        h = (lax.dot_general(agg, wl_ref[...], (((1,), (1,)), ((), ())),
                             preferred_element_type=jnp.float32)
             + lax.dot_general(x_ref[...], wr_ref[...], (((1,), (1,)), ((), ())),
                               preferred_element_type=jnp.float32)
             + b_ref[...])
        o_ref[...] = jnp.maximum(h, 0.0)

    row_spec = pl.BlockSpec((br, d), lambda i: (i, 0))
    return pl.pallas_call(
        body,
        grid=(nb,),
        in_specs=[
            row_spec,
            pl.BlockSpec((br, d), lambda i: (i + off, 0)),
            pl.BlockSpec((br, 128), lambda i: (i, 0)),
            pl.BlockSpec((br, 128), lambda i: (i + off, 0)),
            row_spec,
            pl.BlockSpec((d, d), lambda i: (0, 0)),
            pl.BlockSpec((d, d), lambda i: (0, 0)),
            pl.BlockSpec((1, d), lambda i: (0, 0)),
        ],
        out_specs=row_spec,
        out_shape=jax.ShapeDtypeStruct((n, d), jnp.float32),
    )(parts, parts, cnts, cnts, x, wl, wr, bias.reshape(1, d))


def _tc_head(pparts, pcnts, wc, bias, g, d):
    def body(p0, p1, c0, c1, w_ref, b_ref, o_ref):
        p = p0[...] + p1[...]
        c = c0[...][:, 0:1] + c1[...][:, 0:1]
        pooled = p / jnp.maximum(c, 1.0)
        o_ref[...] = (lax.dot_general(pooled, w_ref[...],
                                      (((1,), (1,)), ((), ())),
                                      preferred_element_type=jnp.float32)
                      + b_ref[...])

    return pl.pallas_call(
        body,
        grid=(1,),
        in_specs=[
            pl.BlockSpec((g, d), lambda i: (0, 0)),
            pl.BlockSpec((g, d), lambda i: (1, 0)),
            pl.BlockSpec((g, 128), lambda i: (0, 0)),
            pl.BlockSpec((g, 128), lambda i: (1, 0)),
            pl.BlockSpec((d, d), lambda i: (0, 0)),
            pl.BlockSpec((1, d), lambda i: (0, 0)),
        ],
        out_specs=pl.BlockSpec((g, d), lambda i: (0, 0)),
        out_shape=jax.ShapeDtypeStruct((g, d), jnp.float32),
    )(pparts, pparts, pcnts, pcnts, wc, bias.reshape(1, d))


def kernel(x, edge_index, batch, W1l, b1l, W1r, W2l, b2l, W2r, Wc, bc):
    g = 64
    d = x.shape[1]
    src = edge_index[0]
    dst = edge_index[1]
    parts1, npad = _sc_edge_aggregate(x, src, dst)
    cnts = _sc_degree_count(dst, x.shape[0])
    h1 = _tc_sage_linear(parts1, cnts, x, W1l, b1l, W1r, npad)
    parts2, npad = _sc_edge_aggregate(h1, src, dst)
    h2 = _tc_sage_linear(parts2, cnts, h1, W2l, b2l, W2r, npad)
    pparts, pcnts = _sc_pool(h2, batch, g)
    return _tc_head(pparts, pcnts, Wc, bc, g, d)
